# baseline (speedup 1.0000x reference)
import functools

import jax
import jax.numpy as jnp
from jax import lax
from jax.experimental import pallas as pl
from jax.experimental.pallas import tpu as pltpu
from jax.experimental.pallas import tpu_sc as plsc

N_NODE = 50001
DIM = 128
E_EDGES = 800000
BATCH = 1024
SLEN = 50
W_K = 12.0

N_PAD = 50176
N_CHUNKS = 8
CHUNK = N_PAD // N_CHUNKS
TILE_ROWS = CHUNK // 16
EPT = E_EDGES // 16
EBLK = 2000
NBLK = EPT // EBLK
ZROWS = 56
CAP1D = 8320
GRP = 128


def _stage_start(rows, cols, vals, off, er, ec, ev, sem):
    pltpu.async_copy(rows.at[pl.ds(off, EBLK)], er, sem)
    pltpu.async_copy(cols.at[pl.ds(off, EBLK)], ec, sem)
    pltpu.async_copy(vals.at[pl.ds(off, EBLK)], ev, sem)


def _stage_wait(rows, cols, vals, off, er, ec, ev, sem):
    pltpu.make_async_copy(rows.at[pl.ds(off, EBLK)], er, sem).wait()
    pltpu.make_async_copy(cols.at[pl.ds(off, EBLK)], ec, sem).wait()
    pltpu.make_async_copy(vals.at[pl.ds(off, EBLK)], ev, sem).wait()


def _sc_scatter_body(emb, rows, cols, vals, h_out,
                     acc, er0, ec0, ev0, er1, ec1, ev1,
                     ccol, clrow, cval,
                     gbuf0, gbuf1, fbuf, idxb,
                     esem0, esem1, gsem0, gsem1, ssem0):
    c = lax.axis_index("c")
    s = lax.axis_index("s")
    tbase = s * EPT
    ebuf = ((er0, ec0, ev0, esem0), (er1, ec1, ev1, esem1))
    ztrue = jnp.ones((16,), jnp.bool_)
    zi = jnp.zeros((16,), jnp.int32)
    zf = jnp.zeros((16,), jnp.float32)

    def _scan_buf(er, ec, ev, cnt):
        def _grp(g, cnt):
            g16 = pl.multiple_of(g * 16, 16)
            r = er[pl.ds(g16, 16)]
            cix = ec[pl.ds(g16, 16)]
            v = ev[pl.ds(g16, 16)]
            m = jnp.logical_and(r >= lo, r < lo + CHUNK)
            plsc.store_compressed(cval.at[pl.ds(cnt, 16)], v, mask=m)
            plsc.store_compressed(ccol.at[pl.ds(cnt, 16)], cix, mask=m)
            plsc.store_compressed(clrow.at[pl.ds(cnt, 16)], r - lo, mask=m)
            return cnt + plsc.all_reduce_population_count(m)[0]
        return lax.fori_loop(0, EBLK // 16, _grp, cnt)

    def _gather_start(j, gb, gs):
        pltpu.async_copy(emb.at[ccol.at[pl.ds(j * GRP, GRP)]], gb, gs)

    def _gather_wait(j, gb, gs):
        pltpu.make_async_copy(emb.at[ccol.at[pl.ds(j * GRP, GRP)]], gb,
                              gs).wait()

    lanes2 = lax.iota(jnp.int32, 16) * 2
    mask_hi = jnp.full((16,), -65536, jnp.int32)

    def _scatter_start(gb, ix, ss):
        pltpu.async_copy(gb, acc.at[ix.at[0]], ss, add=True)

    def _scatter_wait(gb, ix, ss):
        pltpu.make_async_copy(gb, acc.at[ix.at[0]], ss).wait()

    for k in range(N_CHUNKS // 2):
        chunk_id = c * (N_CHUNKS // 2) + k
        lo = chunk_id * CHUNK

        def _zb(r, _):
            for cc in range(8):
                fbuf[r, pl.ds(cc * 16, 16)] = zf
            return 0
        lax.fori_loop(0, ZROWS, _zb, 0)
        for i in range(TILE_ROWS // ZROWS):
            pltpu.sync_copy(fbuf.at[pl.ds(0, ZROWS)],
                            acc.at[pl.ds(s * TILE_ROWS + i * ZROWS, ZROWS)])
        plsc.subcore_barrier()

        _stage_start(rows, cols, vals, pl.multiple_of(tbase, 8), *ebuf[0])

        def _blkpair(bb, cnt):
            blk0 = bb * 2
            off0 = pl.multiple_of(tbase + blk0 * EBLK, 8)
            off1 = pl.multiple_of(tbase + (blk0 + 1) * EBLK, 8)
            off2 = pl.multiple_of(tbase + (blk0 + 2) * EBLK, 8)
            _stage_wait(rows, cols, vals, off0, *ebuf[0])
            _stage_start(rows, cols, vals, off1, *ebuf[1])
            cnt = _scan_buf(er0, ec0, ev0, cnt)
            _stage_wait(rows, cols, vals, off1, *ebuf[1])
            _stage_start(rows, cols, vals, off2, *ebuf[0])
            cnt = _scan_buf(er1, ec1, ev1, cnt)
            return cnt

        cnt = lax.fori_loop(0, (NBLK - 1) // 2, _blkpair, jnp.int32(0))
        lastoff = pl.multiple_of(tbase + (NBLK - 1) * EBLK, 8)
        _stage_wait(rows, cols, vals, lastoff, *ebuf[0])
        cnt = _scan_buf(er0, ec0, ev0, cnt)

        for t in range(GRP // 16):
            po = cnt + t * 16
            plsc.store_compressed(cval.at[pl.ds(po, 16)], zf, mask=ztrue)
            plsc.store_compressed(ccol.at[pl.ds(po, 16)], zi, mask=ztrue)
            plsc.store_compressed(clrow.at[pl.ds(po, 16)], zi, mask=ztrue)

        ng = (cnt + (GRP - 1)) // GRP

        @pl.when(ng > 0)
        def _prologue():
            _gather_start(0, gbuf0, gsem0)

        def _scale_fill(j, gb, fb, ix):
            def _w(rq, _):
                voff = pl.multiple_of(j * GRP + rq * 16, 16)
                ix[0, pl.ds(rq * 16, 16)] = clrow[pl.ds(voff, 16)]
                vblk = cval[pl.ds(voff, 16)]
                for i in range(16):
                    vv = jnp.full((16,), vblk[i], jnp.float32)
                    rr = rq * 16 + i
                    frow = fb.at[rr]
                    for cc in range(4):
                        pair = gb[rr, pl.ds(cc * 16, 16)]
                        ev = plsc.bitcast(lax.shift_left(pair, 16),
                                          jnp.float32)
                        od = plsc.bitcast(lax.bitwise_and(pair, mask_hi),
                                          jnp.float32)
                        plsc.store_scatter(frow, [lanes2 + cc * 32],
                                           ev * vv, mask=None)
                        plsc.store_scatter(frow, [lanes2 + (cc * 32 + 1)],
                                           od * vv, mask=None)
                return 0
            lax.fori_loop(0, GRP // 16, _w, 0)

        def _step(j, b):
            gb, gs = (gbuf0, gsem0) if b == 0 else (gbuf1, gsem1)
            nb, ngs = (gbuf1, gsem1) if b == 0 else (gbuf0, gsem0)

            @pl.when(j + 1 < ng)
            def _():
                _gather_start(j + 1, nb, ngs)

            _gather_wait(j, gb, gs)

            @pl.when(j >= 1)
            def _():
                _scatter_wait(fbuf, idxb, ssem0)

            _scale_fill(j, gb, fbuf, idxb)
            _scatter_start(fbuf, idxb, ssem0)

        def _pair(jj, _):
            j0 = jj * 2

            @pl.when(j0 < ng)
            def _():
                _step(j0, 0)

            @pl.when(j0 + 1 < ng)
            def _():
                _step(j0 + 1, 1)
            return 0

        lax.fori_loop(0, (ng + 1) // 2, _pair, 0)

        @pl.when(ng >= 1)
        def _():
            _scatter_wait(fbuf, idxb, ssem0)

        plsc.subcore_barrier()

        src = pl.multiple_of(s * TILE_ROWS, 8)
        dst = pl.multiple_of(lo + s * TILE_ROWS, 8)
        pltpu.sync_copy(acc.at[pl.ds(src, TILE_ROWS)],
                        h_out.at[pl.ds(dst, TILE_ROWS)])
        plsc.subcore_barrier()


_sc_scatter = functools.partial(
    pl.kernel,
    out_type=jax.ShapeDtypeStruct((N_PAD, DIM), jnp.float32),
    mesh=plsc.VectorSubcoreMesh(core_axis_name="c", subcore_axis_name="s"),
    scratch_types=[
        pltpu.VMEM_SHARED((CHUNK, DIM), jnp.float32),
        pltpu.VMEM((EBLK,), jnp.int32),
        pltpu.VMEM((EBLK,), jnp.int32),
        pltpu.VMEM((EBLK,), jnp.float32),
        pltpu.VMEM((EBLK,), jnp.int32),
        pltpu.VMEM((EBLK,), jnp.int32),
        pltpu.VMEM((EBLK,), jnp.float32),
        pltpu.VMEM((CAP1D,), jnp.int32),
        pltpu.VMEM((CAP1D,), jnp.int32),
        pltpu.VMEM((CAP1D,), jnp.float32),
        pltpu.VMEM((GRP, DIM // 2), jnp.int32),
        pltpu.VMEM((GRP, DIM // 2), jnp.int32),
        pltpu.VMEM((GRP, DIM), jnp.float32),
        pltpu.VMEM((1, 128), jnp.int32),
        pltpu.SemaphoreType.DMA,
        pltpu.SemaphoreType.DMA,
        pltpu.SemaphoreType.DMA,
        pltpu.SemaphoreType.DMA,
        pltpu.SemaphoreType.DMA,
    ],
    compiler_params=pltpu.CompilerParams(needs_layout_passes=False,
                                         use_tc_tiling_on_sc=False),
)(_sc_scatter_body)


GPW = (BATCH * SLEN) // 32


def _sc_gather_body(g, gidx, out, gidxv, rbuf, sem):
    c = lax.axis_index("c")
    s = lax.axis_index("s")
    wid = s * 2 + c
    base = pl.multiple_of(wid * GPW, 8)
    pltpu.sync_copy(gidx.at[pl.ds(base, GPW)], gidxv)
    nfull = GPW // 128
    for k in range(nfull):
        pltpu.async_copy(g.at[gidxv.at[pl.ds(k * 128, 128)]], rbuf, sem).wait()
        pltpu.sync_copy(rbuf, out.at[pl.ds(base + k * 128, 128)])
    tail = GPW - nfull * 128
    if tail:
        pltpu.async_copy(g.at[gidxv.at[pl.ds(nfull * 128, tail)]],
                         rbuf.at[pl.ds(0, tail)], sem).wait()
        pltpu.sync_copy(rbuf.at[pl.ds(0, tail)],
                        out.at[pl.ds(base + nfull * 128, tail)])


_sc_gather = functools.partial(
    pl.kernel,
    out_type=jax.ShapeDtypeStruct((BATCH * SLEN, DIM), jnp.float32),
    mesh=plsc.VectorSubcoreMesh(core_axis_name="c", subcore_axis_name="s"),
    scratch_types=[
        pltpu.VMEM((GPW,), jnp.int32),
        pltpu.VMEM((128, DIM), jnp.float32),
        pltpu.SemaphoreType.DMA,
    ],
)(_sc_gather_body)


BB = 128


def _attn_body(seq_ref, inp_ref, pos_ref, w1_ref, g1w_ref, g1b_ref,
               g2w_ref, w2_ref, o_ref):
    seqh = seq_ref[...].reshape(BB, SLEN, DIM)
    snrm = jnp.sqrt(jnp.sum(seqh * seqh, axis=2, keepdims=True))
    seq = seqh / jnp.maximum(snrm, 1e-12)
    maskf = (inp_ref[...] != 0).astype(jnp.float32)
    msum = jnp.sum(maskf, axis=1, keepdims=True)
    hs = jnp.sum(seq * maskf[:, :, None], axis=1) / msum

    w1 = w1_ref[...]
    posw = jnp.dot(pos_ref[...], w1[0:DIM, :],
                   preferred_element_type=jnp.float32)[0:SLEN]
    sw = jnp.dot(seq.reshape(BB * SLEN, DIM), w1[DIM:2 * DIM, :],
                 preferred_element_type=jnp.float32).reshape(BB, SLEN, DIM)
    nh = jnp.tanh(posw[None, :, :] + sw)

    g2 = lax.dot_general(hs, g2w_ref[...], (((1,), (1,)), ((), ())),
                         preferred_element_type=jnp.float32)
    g1 = lax.dot_general(nh.reshape(BB * SLEN, DIM), g1w_ref[...],
                         (((1,), (1,)), ((), ())),
                         preferred_element_type=jnp.float32).reshape(
                             BB, SLEN, DIM)
    nh2 = jax.nn.sigmoid(g1 + g1b_ref[...][None, :, :] + g2[:, None, :])

    beta = jnp.sum(nh2 * w2_ref[...][None, :, :], axis=-1, keepdims=True)
    beta = beta * maskf[:, :, None]
    sess = jnp.sum(beta * seq, axis=1)
    nrm = jnp.sqrt(jnp.sum(sess * sess, axis=1, keepdims=True))
    o_ref[...] = W_K * sess / jnp.maximum(nrm, 1e-12)


def _attention(seq2d, inp, pos_flip_pad, w_1, glu1_w, glu1_b2, glu2_w, w_2r):
    zero2 = lambda i: (0, 0)
    return pl.pallas_call(
        _attn_body,
        grid=(BATCH // BB,),
        in_specs=[
            pl.BlockSpec((BB, SLEN * DIM), lambda i: (i, 0)),
            pl.BlockSpec((BB, SLEN), lambda i: (i, 0)),
            pl.BlockSpec((64, DIM), zero2),
            pl.BlockSpec((2 * DIM, DIM), zero2),
            pl.BlockSpec((DIM, DIM), zero2),
            pl.BlockSpec((1, DIM), zero2),
            pl.BlockSpec((DIM, DIM), zero2),
            pl.BlockSpec((1, DIM), zero2),
        ],
        out_specs=pl.BlockSpec((BB, DIM), lambda i: (i, 0)),
        out_shape=jax.ShapeDtypeStruct((BATCH, DIM), jnp.float32),
    )(seq2d, inp, pos_flip_pad, w_1, glu1_w, glu1_b2, glu2_w, w_2r)


VB = 1024


def _scores_body(sel_ref, g_ref, o_ref):
    hb = g_ref[...]
    n = jnp.sqrt(jnp.sum(hb * hb, axis=1, keepdims=True))
    hn = hb / jnp.maximum(n, 1e-12)
    o_ref[...] = lax.dot_general(
        sel_ref[...], hn, (((1,), (1,)), ((), ())),
        preferred_element_type=jnp.float32)


def _scores(sel, g):
    nblk = (N_NODE + VB - 1) // VB
    return pl.pallas_call(
        _scores_body,
        grid=(nblk,),
        in_specs=[
            pl.BlockSpec((BATCH, DIM), lambda i: (0, 0)),
            pl.BlockSpec((VB, DIM), lambda i: (i, 0)),
        ],
        out_specs=pl.BlockSpec((BATCH, VB), lambda i: (0, i)),
        out_shape=jax.ShapeDtypeStruct((BATCH, N_NODE), jnp.float32),
    )(sel, g)


def kernel(item_embedding, pos_embedding, w_1, w_2, glu1_w, glu1_b, glu2_w,
           adj_values, items, inputs, alias_inputs, adj_indices):
    rows = adj_indices[0]
    cols = adj_indices[1]

    emb_i32 = lax.bitcast_convert_type(
        item_embedding.astype(jnp.bfloat16).reshape(N_NODE, DIM // 2, 2),
        jnp.int32)
    h = _sc_scatter(emb_i32, rows, cols, adj_values)

    gidx = jnp.take_along_axis(items, alias_inputs, axis=1).reshape(-1)
    seq = _sc_gather(h, gidx.astype(jnp.int32))

    pos_flip_pad = jnp.zeros((64, DIM), jnp.float32).at[0:SLEN].set(
        jnp.flip(pos_embedding, axis=0))
    sel = _attention(seq.reshape(BATCH, SLEN * DIM), inputs,
                     pos_flip_pad, w_1, glu1_w,
                     glu1_b.reshape(1, DIM), glu2_w, w_2.reshape(1, DIM))

    scores = _scores(sel, h)
    return scores, jnp.zeros((1,), jnp.float32)


# speedup vs baseline: 1.4138x; 1.4138x over previous
import functools

import jax
import jax.numpy as jnp
from jax import lax
from jax.experimental import pallas as pl
from jax.experimental.pallas import tpu as pltpu
from jax.experimental.pallas import tpu_sc as plsc

N_NODE = 50001
DIM = 128
E_EDGES = 800000
BATCH = 1024
SLEN = 50
W_K = 12.0

N_PAD = 50176
N_CHUNKS = 8
CHUNK = N_PAD // N_CHUNKS
TILE_ROWS = CHUNK // 16
EPT = E_EDGES // 16
EBLK = 2000
NBLK = EPT // EBLK
ZROWS = 56
CAP1D = 8320
GRP = 128


def _stage_start(rows, cols, vals, off, er, ec, ev, sem):
    pltpu.async_copy(rows.at[pl.ds(off, EBLK)], er, sem)
    pltpu.async_copy(cols.at[pl.ds(off, EBLK)], ec, sem)
    pltpu.async_copy(vals.at[pl.ds(off, EBLK)], ev, sem)


def _stage_wait(rows, cols, vals, off, er, ec, ev, sem):
    pltpu.make_async_copy(rows.at[pl.ds(off, EBLK)], er, sem).wait()
    pltpu.make_async_copy(cols.at[pl.ds(off, EBLK)], ec, sem).wait()
    pltpu.make_async_copy(vals.at[pl.ds(off, EBLK)], ev, sem).wait()


def _sc_scatter_body(emb, rows, cols, vals, h_out,
                     acc, er0, ec0, ev0, er1, ec1, ev1,
                     ccol, clrow, cval,
                     gbuf0, gbuf1, idx0, idx1,
                     esem0, esem1, gsem0, gsem1, ssem0, ssem1):
    c = lax.axis_index("c")
    s = lax.axis_index("s")
    tbase = s * EPT
    ebuf = ((er0, ec0, ev0, esem0), (er1, ec1, ev1, esem1))
    ztrue = jnp.ones((16,), jnp.bool_)
    zi = jnp.zeros((16,), jnp.int32)
    zf = jnp.zeros((16,), jnp.float32)

    def _scan_buf(er, ec, ev, cnt):
        def _grp(g, cnt):
            g16 = pl.multiple_of(g * 16, 16)
            r = er[pl.ds(g16, 16)]
            cix = ec[pl.ds(g16, 16)]
            v = ev[pl.ds(g16, 16)]
            m = jnp.logical_and(r >= lo, r < lo + CHUNK)
            plsc.store_compressed(cval.at[pl.ds(cnt, 16)], v, mask=m)
            plsc.store_compressed(ccol.at[pl.ds(cnt, 16)], cix, mask=m)
            plsc.store_compressed(clrow.at[pl.ds(cnt, 16)], r - lo, mask=m)
            return cnt + plsc.all_reduce_population_count(m)[0]
        return lax.fori_loop(0, EBLK // 16, _grp, cnt)

    def _gather_start(j, gb, gs):
        pltpu.async_copy(emb.at[ccol.at[pl.ds(j * GRP, GRP)]], gb, gs)

    def _gather_wait(j, gb, gs):
        pltpu.make_async_copy(emb.at[ccol.at[pl.ds(j * GRP, GRP)]], gb,
                              gs).wait()

    def _scatter_start(gb, ix, ss):
        pltpu.async_copy(gb, acc.at[ix.at[0]], ss, add=True)

    def _scatter_wait(gb, ix, ss):
        pltpu.make_async_copy(gb, acc.at[ix.at[0]], ss).wait()

    for k in range(N_CHUNKS // 2):
        chunk_id = c * (N_CHUNKS // 2) + k
        lo = chunk_id * CHUNK

        def _zb(r, _):
            for cc in range(8):
                gbuf0[r, pl.ds(cc * 16, 16)] = zf
            return 0
        lax.fori_loop(0, ZROWS, _zb, 0)
        for i in range(TILE_ROWS // ZROWS):
            pltpu.sync_copy(gbuf0.at[pl.ds(0, ZROWS)],
                            acc.at[pl.ds(s * TILE_ROWS + i * ZROWS, ZROWS)])
        plsc.subcore_barrier()

        _stage_start(rows, cols, vals, pl.multiple_of(tbase, 8), *ebuf[0])

        def _blkpair(bb, cnt):
            blk0 = bb * 2
            off0 = pl.multiple_of(tbase + blk0 * EBLK, 8)
            off1 = pl.multiple_of(tbase + (blk0 + 1) * EBLK, 8)
            off2 = pl.multiple_of(tbase + (blk0 + 2) * EBLK, 8)
            _stage_wait(rows, cols, vals, off0, *ebuf[0])
            _stage_start(rows, cols, vals, off1, *ebuf[1])
            cnt = _scan_buf(er0, ec0, ev0, cnt)
            _stage_wait(rows, cols, vals, off1, *ebuf[1])
            _stage_start(rows, cols, vals, off2, *ebuf[0])
            cnt = _scan_buf(er1, ec1, ev1, cnt)
            return cnt

        cnt = lax.fori_loop(0, (NBLK - 1) // 2, _blkpair, jnp.int32(0))
        lastoff = pl.multiple_of(tbase + (NBLK - 1) * EBLK, 8)
        _stage_wait(rows, cols, vals, lastoff, *ebuf[0])
        cnt = _scan_buf(er0, ec0, ev0, cnt)

        for t in range(GRP // 16):
            po = cnt + t * 16
            plsc.store_compressed(cval.at[pl.ds(po, 16)], zf, mask=ztrue)
            plsc.store_compressed(ccol.at[pl.ds(po, 16)], zi, mask=ztrue)
            plsc.store_compressed(clrow.at[pl.ds(po, 16)], zi, mask=ztrue)

        ng = (cnt + (GRP - 1)) // GRP

        @pl.when(ng > 0)
        def _prologue():
            _gather_start(0, gbuf0, gsem0)

        def _scale_fill(j, gb, ix):
            def _w(rq, _):
                voff = pl.multiple_of(j * GRP + rq * 16, 16)
                ix[0, pl.ds(rq * 16, 16)] = clrow[pl.ds(voff, 16)]
                vblk = cval[pl.ds(voff, 16)]
                for i in range(16):
                    vv = jnp.full((16,), vblk[i], jnp.float32)
                    rr = rq * 16 + i
                    for cc in range(8):
                        gb[rr, pl.ds(cc * 16, 16)] = (
                            gb[rr, pl.ds(cc * 16, 16)] * vv)
                return 0
            lax.fori_loop(0, GRP // 16, _w, 0)

        def _step(j, b):
            gb, gs, ix, ss = ((gbuf0, gsem0, idx0, ssem0) if b == 0
                              else (gbuf1, gsem1, idx1, ssem1))
            nb, ngs, nix, nss = ((gbuf1, gsem1, idx1, ssem1) if b == 0
                                 else (gbuf0, gsem0, idx0, ssem0))

            @pl.when(j >= 1)
            def _():
                _scatter_wait(nb, nix, nss)

            @pl.when(j + 1 < ng)
            def _():
                _gather_start(j + 1, nb, ngs)

            _gather_wait(j, gb, gs)
            _scale_fill(j, gb, ix)
            _scatter_start(gb, ix, ss)

        def _pair(jj, _):
            j0 = jj * 2

            @pl.when(j0 < ng)
            def _():
                _step(j0, 0)

            @pl.when(j0 + 1 < ng)
            def _():
                _step(j0 + 1, 1)
            return 0

        lax.fori_loop(0, (ng + 1) // 2, _pair, 0)

        last = ng - 1

        @pl.when(jnp.logical_and(ng >= 1, lax.rem(last, 2) == 0))
        def _():
            _scatter_wait(gbuf0, idx0, ssem0)

        @pl.when(jnp.logical_and(ng >= 1, lax.rem(last, 2) == 1))
        def _():
            _scatter_wait(gbuf1, idx1, ssem1)

        plsc.subcore_barrier()

        src = pl.multiple_of(s * TILE_ROWS, 8)
        dst = pl.multiple_of(lo + s * TILE_ROWS, 8)
        pltpu.sync_copy(acc.at[pl.ds(src, TILE_ROWS)],
                        h_out.at[pl.ds(dst, TILE_ROWS)])
        plsc.subcore_barrier()


_sc_scatter = functools.partial(
    pl.kernel,
    out_type=jax.ShapeDtypeStruct((N_PAD, DIM), jnp.float32),
    mesh=plsc.VectorSubcoreMesh(core_axis_name="c", subcore_axis_name="s"),
    scratch_types=[
        pltpu.VMEM_SHARED((CHUNK, DIM), jnp.float32),
        pltpu.VMEM((EBLK,), jnp.int32),
        pltpu.VMEM((EBLK,), jnp.int32),
        pltpu.VMEM((EBLK,), jnp.float32),
        pltpu.VMEM((EBLK,), jnp.int32),
        pltpu.VMEM((EBLK,), jnp.int32),
        pltpu.VMEM((EBLK,), jnp.float32),
        pltpu.VMEM((CAP1D,), jnp.int32),
        pltpu.VMEM((CAP1D,), jnp.int32),
        pltpu.VMEM((CAP1D,), jnp.float32),
        pltpu.VMEM((GRP, DIM), jnp.float32),
        pltpu.VMEM((GRP, DIM), jnp.float32),
        pltpu.VMEM((1, 128), jnp.int32),
        pltpu.VMEM((1, 128), jnp.int32),
        pltpu.SemaphoreType.DMA,
        pltpu.SemaphoreType.DMA,
        pltpu.SemaphoreType.DMA,
        pltpu.SemaphoreType.DMA,
        pltpu.SemaphoreType.DMA,
        pltpu.SemaphoreType.DMA,
    ],
    compiler_params=pltpu.CompilerParams(needs_layout_passes=False),
)(_sc_scatter_body)


GPW = (BATCH * SLEN) // 32


def _sc_gather_body(g, gidx, out, gidxv, rbuf, sem):
    c = lax.axis_index("c")
    s = lax.axis_index("s")
    wid = s * 2 + c
    base = pl.multiple_of(wid * GPW, 8)
    pltpu.sync_copy(gidx.at[pl.ds(base, GPW)], gidxv)
    nfull = GPW // 128
    for k in range(nfull):
        pltpu.async_copy(g.at[gidxv.at[pl.ds(k * 128, 128)]], rbuf, sem).wait()
        pltpu.sync_copy(rbuf, out.at[pl.ds(base + k * 128, 128)])
    tail = GPW - nfull * 128
    if tail:
        pltpu.async_copy(g.at[gidxv.at[pl.ds(nfull * 128, tail)]],
                         rbuf.at[pl.ds(0, tail)], sem).wait()
        pltpu.sync_copy(rbuf.at[pl.ds(0, tail)],
                        out.at[pl.ds(base + nfull * 128, tail)])


_sc_gather = functools.partial(
    pl.kernel,
    out_type=jax.ShapeDtypeStruct((BATCH * SLEN, DIM), jnp.float32),
    mesh=plsc.VectorSubcoreMesh(core_axis_name="c", subcore_axis_name="s"),
    scratch_types=[
        pltpu.VMEM((GPW,), jnp.int32),
        pltpu.VMEM((128, DIM), jnp.float32),
        pltpu.SemaphoreType.DMA,
    ],
)(_sc_gather_body)


BB = 128


def _attn_body(seq_ref, inp_ref, pos_ref, w1_ref, g1w_ref, g1b_ref,
               g2w_ref, w2_ref, o_ref):
    seqh = seq_ref[...].reshape(BB, SLEN, DIM)
    snrm = jnp.sqrt(jnp.sum(seqh * seqh, axis=2, keepdims=True))
    seq = seqh / jnp.maximum(snrm, 1e-12)
    maskf = (inp_ref[...] != 0).astype(jnp.float32)
    msum = jnp.sum(maskf, axis=1, keepdims=True)
    hs = jnp.sum(seq * maskf[:, :, None], axis=1) / msum

    w1 = w1_ref[...]
    posw = jnp.dot(pos_ref[...], w1[0:DIM, :],
                   preferred_element_type=jnp.float32)[0:SLEN]
    sw = jnp.dot(seq.reshape(BB * SLEN, DIM), w1[DIM:2 * DIM, :],
                 preferred_element_type=jnp.float32).reshape(BB, SLEN, DIM)
    nh = jnp.tanh(posw[None, :, :] + sw)

    g2 = lax.dot_general(hs, g2w_ref[...], (((1,), (1,)), ((), ())),
                         preferred_element_type=jnp.float32)
    g1 = lax.dot_general(nh.reshape(BB * SLEN, DIM), g1w_ref[...],
                         (((1,), (1,)), ((), ())),
                         preferred_element_type=jnp.float32).reshape(
                             BB, SLEN, DIM)
    nh2 = jax.nn.sigmoid(g1 + g1b_ref[...][None, :, :] + g2[:, None, :])

    beta = jnp.sum(nh2 * w2_ref[...][None, :, :], axis=-1, keepdims=True)
    beta = beta * maskf[:, :, None]
    sess = jnp.sum(beta * seq, axis=1)
    nrm = jnp.sqrt(jnp.sum(sess * sess, axis=1, keepdims=True))
    o_ref[...] = W_K * sess / jnp.maximum(nrm, 1e-12)


def _attention(seq2d, inp, pos_flip_pad, w_1, glu1_w, glu1_b2, glu2_w, w_2r):
    zero2 = lambda i: (0, 0)
    return pl.pallas_call(
        _attn_body,
        grid=(BATCH // BB,),
        in_specs=[
            pl.BlockSpec((BB, SLEN * DIM), lambda i: (i, 0)),
            pl.BlockSpec((BB, SLEN), lambda i: (i, 0)),
            pl.BlockSpec((64, DIM), zero2),
            pl.BlockSpec((2 * DIM, DIM), zero2),
            pl.BlockSpec((DIM, DIM), zero2),
            pl.BlockSpec((1, DIM), zero2),
            pl.BlockSpec((DIM, DIM), zero2),
            pl.BlockSpec((1, DIM), zero2),
        ],
        out_specs=pl.BlockSpec((BB, DIM), lambda i: (i, 0)),
        out_shape=jax.ShapeDtypeStruct((BATCH, DIM), jnp.float32),
    )(seq2d, inp, pos_flip_pad, w_1, glu1_w, glu1_b2, glu2_w, w_2r)


VB = 1024


def _scores_body(sel_ref, g_ref, o_ref):
    hb = g_ref[...]
    n = jnp.sqrt(jnp.sum(hb * hb, axis=1, keepdims=True))
    hn = hb / jnp.maximum(n, 1e-12)
    o_ref[...] = lax.dot_general(
        sel_ref[...], hn, (((1,), (1,)), ((), ())),
        preferred_element_type=jnp.float32)


def _scores(sel, g):
    nblk = (N_NODE + VB - 1) // VB
    return pl.pallas_call(
        _scores_body,
        grid=(nblk,),
        in_specs=[
            pl.BlockSpec((BATCH, DIM), lambda i: (0, 0)),
            pl.BlockSpec((VB, DIM), lambda i: (i, 0)),
        ],
        out_specs=pl.BlockSpec((BATCH, VB), lambda i: (0, i)),
        out_shape=jax.ShapeDtypeStruct((BATCH, N_NODE), jnp.float32),
    )(sel, g)


def kernel(item_embedding, pos_embedding, w_1, w_2, glu1_w, glu1_b, glu2_w,
           adj_values, items, inputs, alias_inputs, adj_indices):
    rows = adj_indices[0]
    cols = adj_indices[1]

    h = _sc_scatter(item_embedding, rows, cols, adj_values)

    gidx = jnp.take_along_axis(items, alias_inputs, axis=1).reshape(-1)
    seq = _sc_gather(h, gidx.astype(jnp.int32))

    pos_flip_pad = jnp.zeros((64, DIM), jnp.float32).at[0:SLEN].set(
        jnp.flip(pos_embedding, axis=0))
    sel = _attention(seq.reshape(BATCH, SLEN * DIM), inputs,
                     pos_flip_pad, w_1, glu1_w,
                     glu1_b.reshape(1, DIM), glu2_w, w_2.reshape(1, DIM))

    scores = _scores(sel, h)
    return scores, jnp.zeros((1,), jnp.float32)


# speedup vs baseline: 1.6369x; 1.1578x over previous
import functools

import jax
import jax.numpy as jnp
from jax import lax
from jax.experimental import pallas as pl
from jax.experimental.pallas import tpu as pltpu
from jax.experimental.pallas import tpu_sc as plsc

N_NODE = 50001
DIM = 128
E_EDGES = 800000
BATCH = 1024
SLEN = 50
W_K = 12.0

N_PAD = 50176
N_CHUNKS = 8
CHUNK = N_PAD // N_CHUNKS
TILE_ROWS = CHUNK // 16
EPT = E_EDGES // 16
EBLK = 2000
NBLK = EPT // EBLK
ZROWS = 56
CAP1D = 8320
GRP = 64


def _stage_start(rows, cols, vals, off, er, ec, ev, sem):
    pltpu.async_copy(rows.at[pl.ds(off, EBLK)], er, sem)
    pltpu.async_copy(cols.at[pl.ds(off, EBLK)], ec, sem)
    pltpu.async_copy(vals.at[pl.ds(off, EBLK)], ev, sem)


def _stage_wait(rows, cols, vals, off, er, ec, ev, sem):
    pltpu.make_async_copy(rows.at[pl.ds(off, EBLK)], er, sem).wait()
    pltpu.make_async_copy(cols.at[pl.ds(off, EBLK)], ec, sem).wait()
    pltpu.make_async_copy(vals.at[pl.ds(off, EBLK)], ev, sem).wait()


def _sc_scatter_body(emb, rows, cols, vals, h_out,
                     acc, er0, ec0, ev0, er1, ec1, ev1,
                     ccol, clrow, cval,
                     gbuf0, gbuf1, gbuf2, gbuf3, fbuf, idxb,
                     esem0, esem1, gsem0, gsem1, gsem2, gsem3, ssem0):
    c = lax.axis_index("c")
    s = lax.axis_index("s")
    tbase = s * EPT
    ebuf = ((er0, ec0, ev0, esem0), (er1, ec1, ev1, esem1))
    ztrue = jnp.ones((16,), jnp.bool_)
    zi = jnp.zeros((16,), jnp.int32)
    zf = jnp.zeros((16,), jnp.float32)

    def _scan_buf(er, ec, ev, cnt):
        def _grp(g, cnt):
            g16 = pl.multiple_of(g * 16, 16)
            r = er[pl.ds(g16, 16)]
            cix = ec[pl.ds(g16, 16)]
            v = ev[pl.ds(g16, 16)]
            m = jnp.logical_and(r >= lo, r < lo + CHUNK)
            plsc.store_compressed(cval.at[pl.ds(cnt, 16)], v, mask=m)
            plsc.store_compressed(ccol.at[pl.ds(cnt, 16)], cix, mask=m)
            plsc.store_compressed(clrow.at[pl.ds(cnt, 16)], r - lo, mask=m)
            return cnt + plsc.all_reduce_population_count(m)[0]
        return lax.fori_loop(0, EBLK // 16, _grp, cnt)

    def _gather_start(j, gb, gs):
        pltpu.async_copy(emb.at[ccol.at[pl.ds(j * GRP, GRP)]], gb, gs)

    def _gather_wait(j, gb, gs):
        pltpu.make_async_copy(emb.at[ccol.at[pl.ds(j * GRP, GRP)]], gb,
                              gs).wait()

    def _scatter_start(gb, ix, ss):
        pltpu.async_copy(gb, acc.at[ix.at[0]], ss, add=True)

    def _scatter_wait(gb, ix, ss):
        pltpu.make_async_copy(gb, acc.at[ix.at[0]], ss).wait()

    for k in range(N_CHUNKS // 2):
        chunk_id = c * (N_CHUNKS // 2) + k
        lo = chunk_id * CHUNK

        def _zb(r, _):
            for cc in range(8):
                fbuf[r, pl.ds(cc * 16, 16)] = zf
            return 0
        lax.fori_loop(0, ZROWS, _zb, 0)
        for i in range(TILE_ROWS // ZROWS):
            pltpu.sync_copy(fbuf.at[pl.ds(0, ZROWS)],
                            acc.at[pl.ds(s * TILE_ROWS + i * ZROWS, ZROWS)])
        plsc.subcore_barrier()

        _stage_start(rows, cols, vals, pl.multiple_of(tbase, 8), *ebuf[0])

        def _blkpair(bb, cnt):
            blk0 = bb * 2
            off0 = pl.multiple_of(tbase + blk0 * EBLK, 8)
            off1 = pl.multiple_of(tbase + (blk0 + 1) * EBLK, 8)
            off2 = pl.multiple_of(tbase + (blk0 + 2) * EBLK, 8)
            _stage_wait(rows, cols, vals, off0, *ebuf[0])
            _stage_start(rows, cols, vals, off1, *ebuf[1])
            cnt = _scan_buf(er0, ec0, ev0, cnt)
            _stage_wait(rows, cols, vals, off1, *ebuf[1])
            _stage_start(rows, cols, vals, off2, *ebuf[0])
            cnt = _scan_buf(er1, ec1, ev1, cnt)
            return cnt

        cnt = lax.fori_loop(0, (NBLK - 1) // 2, _blkpair, jnp.int32(0))
        lastoff = pl.multiple_of(tbase + (NBLK - 1) * EBLK, 8)
        _stage_wait(rows, cols, vals, lastoff, *ebuf[0])
        cnt = _scan_buf(er0, ec0, ev0, cnt)

        for t in range(GRP // 16):
            po = cnt + t * 16
            plsc.store_compressed(cval.at[pl.ds(po, 16)], zf, mask=ztrue)
            plsc.store_compressed(ccol.at[pl.ds(po, 16)], zi, mask=ztrue)
            plsc.store_compressed(clrow.at[pl.ds(po, 16)], zi, mask=ztrue)

        ng = (cnt + (GRP - 1)) // GRP

        gbs = ((gbuf0, gsem0), (gbuf1, gsem1), (gbuf2, gsem2),
               (gbuf3, gsem3))
        for pj in range(3):
            @pl.when(pj < ng)
            def _(pj=pj):
                _gather_start(pj, *gbs[pj])

        def _scale_fill(j, gb, ix):
            def _w(rq, _):
                voff = pl.multiple_of(j * GRP + rq * 16, 16)
                ix[0, pl.ds(rq * 16, 16)] = clrow[pl.ds(voff, 16)]
                vblk = cval[pl.ds(voff, 16)]
                for i in range(16):
                    vv = jnp.full((16,), vblk[i], jnp.float32)
                    rr = rq * 16 + i
                    for cc in range(8):
                        fbuf[rr, pl.ds(cc * 16, 16)] = (
                            gb[rr, pl.ds(cc * 16, 16)] * vv)
                return 0
            lax.fori_loop(0, GRP // 16, _w, 0)

        def _step(j, b):
            gb, gs = gbs[b]
            pb, ps = gbs[(b + 3) % 4]

            @pl.when(j + 3 < ng)
            def _():
                _gather_start(j + 3, pb, ps)

            _gather_wait(j, gb, gs)

            @pl.when(j >= 1)
            def _():
                _scatter_wait(fbuf, idxb, ssem0)

            _scale_fill(j, gb, idxb)
            _scatter_start(fbuf, idxb, ssem0)

        def _quad(jj, _):
            j0 = jj * 4
            for b in range(4):
                @pl.when(j0 + b < ng)
                def _(b=b):
                    _step(j0 + b, b)
            return 0

        lax.fori_loop(0, (ng + 3) // 4, _quad, 0)

        @pl.when(ng >= 1)
        def _():
            _scatter_wait(fbuf, idxb, ssem0)

        plsc.subcore_barrier()

        src = pl.multiple_of(s * TILE_ROWS, 8)
        dst = pl.multiple_of(lo + s * TILE_ROWS, 8)
        pltpu.sync_copy(acc.at[pl.ds(src, TILE_ROWS)],
                        h_out.at[pl.ds(dst, TILE_ROWS)])
        plsc.subcore_barrier()


_sc_scatter = functools.partial(
    pl.kernel,
    out_type=jax.ShapeDtypeStruct((N_PAD, DIM), jnp.float32),
    mesh=plsc.VectorSubcoreMesh(core_axis_name="c", subcore_axis_name="s"),
    scratch_types=[
        pltpu.VMEM_SHARED((CHUNK, DIM), jnp.float32),
        pltpu.VMEM((EBLK,), jnp.int32),
        pltpu.VMEM((EBLK,), jnp.int32),
        pltpu.VMEM((EBLK,), jnp.float32),
        pltpu.VMEM((EBLK,), jnp.int32),
        pltpu.VMEM((EBLK,), jnp.int32),
        pltpu.VMEM((EBLK,), jnp.float32),
        pltpu.VMEM((CAP1D,), jnp.int32),
        pltpu.VMEM((CAP1D,), jnp.int32),
        pltpu.VMEM((CAP1D,), jnp.float32),
        pltpu.VMEM((GRP, DIM), jnp.float32),
        pltpu.VMEM((GRP, DIM), jnp.float32),
        pltpu.VMEM((GRP, DIM), jnp.float32),
        pltpu.VMEM((GRP, DIM), jnp.float32),
        pltpu.VMEM((GRP, DIM), jnp.float32),
        pltpu.VMEM((1, GRP), jnp.int32),
        pltpu.SemaphoreType.DMA,
        pltpu.SemaphoreType.DMA,
        pltpu.SemaphoreType.DMA,
        pltpu.SemaphoreType.DMA,
        pltpu.SemaphoreType.DMA,
        pltpu.SemaphoreType.DMA,
        pltpu.SemaphoreType.DMA,
    ],
    compiler_params=pltpu.CompilerParams(needs_layout_passes=False),
)(_sc_scatter_body)


GPW = (BATCH * SLEN) // 32


def _sc_gather_body(g, gidx, out, gidxv, rbuf, sem):
    c = lax.axis_index("c")
    s = lax.axis_index("s")
    wid = s * 2 + c
    base = pl.multiple_of(wid * GPW, 8)
    pltpu.sync_copy(gidx.at[pl.ds(base, GPW)], gidxv)
    nfull = GPW // 128
    for k in range(nfull):
        pltpu.async_copy(g.at[gidxv.at[pl.ds(k * 128, 128)]], rbuf, sem).wait()
        pltpu.sync_copy(rbuf, out.at[pl.ds(base + k * 128, 128)])
    tail = GPW - nfull * 128
    if tail:
        pltpu.async_copy(g.at[gidxv.at[pl.ds(nfull * 128, tail)]],
                         rbuf.at[pl.ds(0, tail)], sem).wait()
        pltpu.sync_copy(rbuf.at[pl.ds(0, tail)],
                        out.at[pl.ds(base + nfull * 128, tail)])


_sc_gather = functools.partial(
    pl.kernel,
    out_type=jax.ShapeDtypeStruct((BATCH * SLEN, DIM), jnp.float32),
    mesh=plsc.VectorSubcoreMesh(core_axis_name="c", subcore_axis_name="s"),
    scratch_types=[
        pltpu.VMEM((GPW,), jnp.int32),
        pltpu.VMEM((128, DIM), jnp.float32),
        pltpu.SemaphoreType.DMA,
    ],
)(_sc_gather_body)


BB = 128


def _attn_body(seq_ref, inp_ref, pos_ref, w1_ref, g1w_ref, g1b_ref,
               g2w_ref, w2_ref, o_ref):
    seqh = seq_ref[...].reshape(BB, SLEN, DIM)
    snrm = jnp.sqrt(jnp.sum(seqh * seqh, axis=2, keepdims=True))
    seq = seqh / jnp.maximum(snrm, 1e-12)
    maskf = (inp_ref[...] != 0).astype(jnp.float32)
    msum = jnp.sum(maskf, axis=1, keepdims=True)
    hs = jnp.sum(seq * maskf[:, :, None], axis=1) / msum

    w1 = w1_ref[...]
    posw = jnp.dot(pos_ref[...], w1[0:DIM, :],
                   preferred_element_type=jnp.float32)[0:SLEN]
    sw = jnp.dot(seq.reshape(BB * SLEN, DIM), w1[DIM:2 * DIM, :],
                 preferred_element_type=jnp.float32).reshape(BB, SLEN, DIM)
    nh = jnp.tanh(posw[None, :, :] + sw)

    g2 = lax.dot_general(hs, g2w_ref[...], (((1,), (1,)), ((), ())),
                         preferred_element_type=jnp.float32)
    g1 = lax.dot_general(nh.reshape(BB * SLEN, DIM), g1w_ref[...],
                         (((1,), (1,)), ((), ())),
                         preferred_element_type=jnp.float32).reshape(
                             BB, SLEN, DIM)
    nh2 = jax.nn.sigmoid(g1 + g1b_ref[...][None, :, :] + g2[:, None, :])

    beta = jnp.sum(nh2 * w2_ref[...][None, :, :], axis=-1, keepdims=True)
    beta = beta * maskf[:, :, None]
    sess = jnp.sum(beta * seq, axis=1)
    nrm = jnp.sqrt(jnp.sum(sess * sess, axis=1, keepdims=True))
    o_ref[...] = W_K * sess / jnp.maximum(nrm, 1e-12)


def _attention(seq2d, inp, pos_flip_pad, w_1, glu1_w, glu1_b2, glu2_w, w_2r):
    zero2 = lambda i: (0, 0)
    return pl.pallas_call(
        _attn_body,
        grid=(BATCH // BB,),
        in_specs=[
            pl.BlockSpec((BB, SLEN * DIM), lambda i: (i, 0)),
            pl.BlockSpec((BB, SLEN), lambda i: (i, 0)),
            pl.BlockSpec((64, DIM), zero2),
            pl.BlockSpec((2 * DIM, DIM), zero2),
            pl.BlockSpec((DIM, DIM), zero2),
            pl.BlockSpec((1, DIM), zero2),
            pl.BlockSpec((DIM, DIM), zero2),
            pl.BlockSpec((1, DIM), zero2),
        ],
        out_specs=pl.BlockSpec((BB, DIM), lambda i: (i, 0)),
        out_shape=jax.ShapeDtypeStruct((BATCH, DIM), jnp.float32),
    )(seq2d, inp, pos_flip_pad, w_1, glu1_w, glu1_b2, glu2_w, w_2r)


VB = 1024


def _scores_body(sel_ref, g_ref, o_ref):
    hb = g_ref[...]
    n = jnp.sqrt(jnp.sum(hb * hb, axis=1, keepdims=True))
    hn = hb / jnp.maximum(n, 1e-12)
    o_ref[...] = lax.dot_general(
        sel_ref[...], hn, (((1,), (1,)), ((), ())),
        preferred_element_type=jnp.float32)


def _scores(sel, g):
    nblk = (N_NODE + VB - 1) // VB
    return pl.pallas_call(
        _scores_body,
        grid=(nblk,),
        in_specs=[
            pl.BlockSpec((BATCH, DIM), lambda i: (0, 0)),
            pl.BlockSpec((VB, DIM), lambda i: (i, 0)),
        ],
        out_specs=pl.BlockSpec((BATCH, VB), lambda i: (0, i)),
        out_shape=jax.ShapeDtypeStruct((BATCH, N_NODE), jnp.float32),
    )(sel, g)


def kernel(item_embedding, pos_embedding, w_1, w_2, glu1_w, glu1_b, glu2_w,
           adj_values, items, inputs, alias_inputs, adj_indices):
    rows = adj_indices[0]
    cols = adj_indices[1]

    h = _sc_scatter(item_embedding, rows, cols, adj_values)

    gidx = jnp.take_along_axis(items, alias_inputs, axis=1).reshape(-1)
    seq = _sc_gather(h, gidx.astype(jnp.int32))

    pos_flip_pad = jnp.zeros((64, DIM), jnp.float32).at[0:SLEN].set(
        jnp.flip(pos_embedding, axis=0))
    sel = _attention(seq.reshape(BATCH, SLEN * DIM), inputs,
                     pos_flip_pad, w_1, glu1_w,
                     glu1_b.reshape(1, DIM), glu2_w, w_2.reshape(1, DIM))

    scores = _scores(sel, h)
    return scores, jnp.zeros((1,), jnp.float32)


# speedup vs baseline: 1.6485x; 1.0071x over previous
import functools

import jax
import jax.numpy as jnp
from jax import lax
from jax.experimental import pallas as pl
from jax.experimental.pallas import tpu as pltpu
from jax.experimental.pallas import tpu_sc as plsc

N_NODE = 50001
DIM = 128
E_EDGES = 800000
BATCH = 1024
SLEN = 50
W_K = 12.0

N_PAD = 50176
N_CHUNKS = 8
CHUNK = N_PAD // N_CHUNKS
TILE_ROWS = CHUNK // 16
EPT = E_EDGES // 16
EBLK = 2000
NBLK = EPT // EBLK
ZROWS = 56
CAP1D = 8320
GRP = 64


def _stage_start(rows, cols, vals, off, er, ec, ev, sem):
    pltpu.async_copy(rows.at[pl.ds(off, EBLK)], er, sem)
    pltpu.async_copy(cols.at[pl.ds(off, EBLK)], ec, sem)
    pltpu.async_copy(vals.at[pl.ds(off, EBLK)], ev, sem)


def _stage_wait(rows, cols, vals, off, er, ec, ev, sem):
    pltpu.make_async_copy(rows.at[pl.ds(off, EBLK)], er, sem).wait()
    pltpu.make_async_copy(cols.at[pl.ds(off, EBLK)], ec, sem).wait()
    pltpu.make_async_copy(vals.at[pl.ds(off, EBLK)], ev, sem).wait()


def _sc_scatter_body(emb, rows, cols, vals, h_out,
                     acc, er0, ec0, ev0, er1, ec1, ev1,
                     ccol, clrow, cval,
                     gbuf0, gbuf1, gbuf2, gbuf3, fbuf, idxb,
                     esem0, esem1, gsem0, gsem1, gsem2, gsem3, ssem0):
    c = lax.axis_index("c")
    s = lax.axis_index("s")
    tbase = s * EPT
    ebuf = ((er0, ec0, ev0, esem0), (er1, ec1, ev1, esem1))
    ztrue = jnp.ones((16,), jnp.bool_)
    zi = jnp.zeros((16,), jnp.int32)
    zf = jnp.zeros((16,), jnp.float32)

    def _scan_buf(er, ec, ev, cnt):
        def _grp(g, cnt):
            for u in range(2):
                g16 = pl.multiple_of(g * 32 + u * 16, 16)
                r = er[pl.ds(g16, 16)]
                cix = ec[pl.ds(g16, 16)]
                v = ev[pl.ds(g16, 16)]
                m = jnp.logical_and(r >= lo, r < lo + CHUNK)
                plsc.store_compressed(cval.at[pl.ds(cnt, 16)], v, mask=m)
                plsc.store_compressed(ccol.at[pl.ds(cnt, 16)], cix, mask=m)
                plsc.store_compressed(clrow.at[pl.ds(cnt, 16)], r - lo,
                                      mask=m)
                cnt = cnt + plsc.all_reduce_population_count(m)[0]
            return cnt
        return lax.fori_loop(0, EBLK // 32, _grp, cnt)

    def _gather_start(j, gb, gs):
        pltpu.async_copy(emb.at[ccol.at[pl.ds(j * GRP, GRP)]], gb, gs)

    def _gather_wait(j, gb, gs):
        pltpu.make_async_copy(emb.at[ccol.at[pl.ds(j * GRP, GRP)]], gb,
                              gs).wait()

    def _scatter_start(gb, ix, ss):
        pltpu.async_copy(gb, acc.at[ix.at[0]], ss, add=True)

    def _scatter_wait(gb, ix, ss):
        pltpu.make_async_copy(gb, acc.at[ix.at[0]], ss).wait()

    for k in range(N_CHUNKS // 2):
        chunk_id = c * (N_CHUNKS // 2) + k
        lo = chunk_id * CHUNK

        def _zb(r, _):
            for cc in range(8):
                fbuf[r, pl.ds(cc * 16, 16)] = zf
            return 0
        lax.fori_loop(0, ZROWS, _zb, 0)
        for i in range(TILE_ROWS // ZROWS):
            pltpu.sync_copy(fbuf.at[pl.ds(0, ZROWS)],
                            acc.at[pl.ds(s * TILE_ROWS + i * ZROWS, ZROWS)])
        plsc.subcore_barrier()

        _stage_start(rows, cols, vals, pl.multiple_of(tbase, 8), *ebuf[0])

        def _blkpair(bb, cnt):
            blk0 = bb * 2
            off0 = pl.multiple_of(tbase + blk0 * EBLK, 8)
            off1 = pl.multiple_of(tbase + (blk0 + 1) * EBLK, 8)
            off2 = pl.multiple_of(tbase + (blk0 + 2) * EBLK, 8)
            _stage_wait(rows, cols, vals, off0, *ebuf[0])
            _stage_start(rows, cols, vals, off1, *ebuf[1])
            cnt = _scan_buf(er0, ec0, ev0, cnt)
            _stage_wait(rows, cols, vals, off1, *ebuf[1])
            _stage_start(rows, cols, vals, off2, *ebuf[0])
            cnt = _scan_buf(er1, ec1, ev1, cnt)
            return cnt

        cnt = lax.fori_loop(0, (NBLK - 1) // 2, _blkpair, jnp.int32(0))
        lastoff = pl.multiple_of(tbase + (NBLK - 1) * EBLK, 8)
        _stage_wait(rows, cols, vals, lastoff, *ebuf[0])
        cnt = _scan_buf(er0, ec0, ev0, cnt)

        for t in range(GRP // 16):
            po = cnt + t * 16
            plsc.store_compressed(cval.at[pl.ds(po, 16)], zf, mask=ztrue)
            plsc.store_compressed(ccol.at[pl.ds(po, 16)], zi, mask=ztrue)
            plsc.store_compressed(clrow.at[pl.ds(po, 16)], zi, mask=ztrue)

        ng = (cnt + (GRP - 1)) // GRP

        gbs = ((gbuf0, gsem0), (gbuf1, gsem1), (gbuf2, gsem2),
               (gbuf3, gsem3))
        for pj in range(3):
            @pl.when(pj < ng)
            def _(pj=pj):
                _gather_start(pj, *gbs[pj])

        def _scale_fill(j, gb, ix):
            def _w(rq, _):
                voff = pl.multiple_of(j * GRP + rq * 16, 16)
                ix[0, pl.ds(rq * 16, 16)] = clrow[pl.ds(voff, 16)]
                vblk = cval[pl.ds(voff, 16)]
                for i in range(16):
                    vv = jnp.full((16,), vblk[i], jnp.float32)
                    rr = rq * 16 + i
                    for cc in range(8):
                        fbuf[rr, pl.ds(cc * 16, 16)] = (
                            gb[rr, pl.ds(cc * 16, 16)] * vv)
                return 0
            lax.fori_loop(0, GRP // 16, _w, 0)

        def _step(j, b):
            gb, gs = gbs[b]
            pb, ps = gbs[(b + 3) % 4]

            @pl.when(j + 3 < ng)
            def _():
                _gather_start(j + 3, pb, ps)

            _gather_wait(j, gb, gs)

            @pl.when(j >= 1)
            def _():
                _scatter_wait(fbuf, idxb, ssem0)

            _scale_fill(j, gb, idxb)
            _scatter_start(fbuf, idxb, ssem0)

        def _quad(jj, _):
            j0 = jj * 4
            for b in range(4):
                @pl.when(j0 + b < ng)
                def _(b=b):
                    _step(j0 + b, b)
            return 0

        lax.fori_loop(0, (ng + 3) // 4, _quad, 0)

        @pl.when(ng >= 1)
        def _():
            _scatter_wait(fbuf, idxb, ssem0)

        plsc.subcore_barrier()

        src = pl.multiple_of(s * TILE_ROWS, 8)
        dst = pl.multiple_of(lo + s * TILE_ROWS, 8)
        pltpu.sync_copy(acc.at[pl.ds(src, TILE_ROWS)],
                        h_out.at[pl.ds(dst, TILE_ROWS)])
        plsc.subcore_barrier()


_sc_scatter = functools.partial(
    pl.kernel,
    out_type=jax.ShapeDtypeStruct((N_PAD, DIM), jnp.float32),
    mesh=plsc.VectorSubcoreMesh(core_axis_name="c", subcore_axis_name="s"),
    scratch_types=[
        pltpu.VMEM_SHARED((CHUNK, DIM), jnp.float32),
        pltpu.VMEM((EBLK,), jnp.int32),
        pltpu.VMEM((EBLK,), jnp.int32),
        pltpu.VMEM((EBLK,), jnp.float32),
        pltpu.VMEM((EBLK,), jnp.int32),
        pltpu.VMEM((EBLK,), jnp.int32),
        pltpu.VMEM((EBLK,), jnp.float32),
        pltpu.VMEM((CAP1D,), jnp.int32),
        pltpu.VMEM((CAP1D,), jnp.int32),
        pltpu.VMEM((CAP1D,), jnp.float32),
        pltpu.VMEM((GRP, DIM), jnp.float32),
        pltpu.VMEM((GRP, DIM), jnp.float32),
        pltpu.VMEM((GRP, DIM), jnp.float32),
        pltpu.VMEM((GRP, DIM), jnp.float32),
        pltpu.VMEM((GRP, DIM), jnp.float32),
        pltpu.VMEM((1, GRP), jnp.int32),
        pltpu.SemaphoreType.DMA,
        pltpu.SemaphoreType.DMA,
        pltpu.SemaphoreType.DMA,
        pltpu.SemaphoreType.DMA,
        pltpu.SemaphoreType.DMA,
        pltpu.SemaphoreType.DMA,
        pltpu.SemaphoreType.DMA,
    ],
    compiler_params=pltpu.CompilerParams(needs_layout_passes=False),
)(_sc_scatter_body)


GPW = (BATCH * SLEN) // 32


def _sc_gather_body(g, gidx, out, gidxv, rbuf, sem):
    c = lax.axis_index("c")
    s = lax.axis_index("s")
    wid = s * 2 + c
    base = pl.multiple_of(wid * GPW, 8)
    pltpu.sync_copy(gidx.at[pl.ds(base, GPW)], gidxv)
    nfull = GPW // 128
    for k in range(nfull):
        pltpu.async_copy(g.at[gidxv.at[pl.ds(k * 128, 128)]], rbuf, sem).wait()
        pltpu.sync_copy(rbuf, out.at[pl.ds(base + k * 128, 128)])
    tail = GPW - nfull * 128
    if tail:
        pltpu.async_copy(g.at[gidxv.at[pl.ds(nfull * 128, tail)]],
                         rbuf.at[pl.ds(0, tail)], sem).wait()
        pltpu.sync_copy(rbuf.at[pl.ds(0, tail)],
                        out.at[pl.ds(base + nfull * 128, tail)])


_sc_gather = functools.partial(
    pl.kernel,
    out_type=jax.ShapeDtypeStruct((BATCH * SLEN, DIM), jnp.float32),
    mesh=plsc.VectorSubcoreMesh(core_axis_name="c", subcore_axis_name="s"),
    scratch_types=[
        pltpu.VMEM((GPW,), jnp.int32),
        pltpu.VMEM((128, DIM), jnp.float32),
        pltpu.SemaphoreType.DMA,
    ],
)(_sc_gather_body)


BB = 128


def _attn_body(seq_ref, inp_ref, pos_ref, w1_ref, g1w_ref, g1b_ref,
               g2w_ref, w2_ref, o_ref):
    seqh = seq_ref[...].reshape(BB, SLEN, DIM)
    snrm = jnp.sqrt(jnp.sum(seqh * seqh, axis=2, keepdims=True))
    seq = seqh / jnp.maximum(snrm, 1e-12)
    maskf = (inp_ref[...] != 0).astype(jnp.float32)
    msum = jnp.sum(maskf, axis=1, keepdims=True)
    hs = jnp.sum(seq * maskf[:, :, None], axis=1) / msum

    w1 = w1_ref[...]
    posw = jnp.dot(pos_ref[...], w1[0:DIM, :],
                   preferred_element_type=jnp.float32)[0:SLEN]
    sw = jnp.dot(seq.reshape(BB * SLEN, DIM), w1[DIM:2 * DIM, :],
                 preferred_element_type=jnp.float32).reshape(BB, SLEN, DIM)
    nh = jnp.tanh(posw[None, :, :] + sw)

    g2 = lax.dot_general(hs, g2w_ref[...], (((1,), (1,)), ((), ())),
                         preferred_element_type=jnp.float32)
    g1 = lax.dot_general(nh.reshape(BB * SLEN, DIM), g1w_ref[...],
                         (((1,), (1,)), ((), ())),
                         preferred_element_type=jnp.float32).reshape(
                             BB, SLEN, DIM)
    nh2 = jax.nn.sigmoid(g1 + g1b_ref[...][None, :, :] + g2[:, None, :])

    beta = jnp.sum(nh2 * w2_ref[...][None, :, :], axis=-1, keepdims=True)
    beta = beta * maskf[:, :, None]
    sess = jnp.sum(beta * seq, axis=1)
    nrm = jnp.sqrt(jnp.sum(sess * sess, axis=1, keepdims=True))
    o_ref[...] = W_K * sess / jnp.maximum(nrm, 1e-12)


def _attention(seq2d, inp, pos_flip_pad, w_1, glu1_w, glu1_b2, glu2_w, w_2r):
    zero2 = lambda i: (0, 0)
    return pl.pallas_call(
        _attn_body,
        grid=(BATCH // BB,),
        in_specs=[
            pl.BlockSpec((BB, SLEN * DIM), lambda i: (i, 0)),
            pl.BlockSpec((BB, SLEN), lambda i: (i, 0)),
            pl.BlockSpec((64, DIM), zero2),
            pl.BlockSpec((2 * DIM, DIM), zero2),
            pl.BlockSpec((DIM, DIM), zero2),
            pl.BlockSpec((1, DIM), zero2),
            pl.BlockSpec((DIM, DIM), zero2),
            pl.BlockSpec((1, DIM), zero2),
        ],
        out_specs=pl.BlockSpec((BB, DIM), lambda i: (i, 0)),
        out_shape=jax.ShapeDtypeStruct((BATCH, DIM), jnp.float32),
    )(seq2d, inp, pos_flip_pad, w_1, glu1_w, glu1_b2, glu2_w, w_2r)


VB = 1024


def _scores_body(sel_ref, g_ref, o_ref):
    hb = g_ref[...]
    n = jnp.sqrt(jnp.sum(hb * hb, axis=1, keepdims=True))
    hn = hb / jnp.maximum(n, 1e-12)
    o_ref[...] = lax.dot_general(
        sel_ref[...], hn, (((1,), (1,)), ((), ())),
        preferred_element_type=jnp.float32)


def _scores(sel, g):
    nblk = (N_NODE + VB - 1) // VB
    return pl.pallas_call(
        _scores_body,
        grid=(nblk,),
        in_specs=[
            pl.BlockSpec((BATCH, DIM), lambda i: (0, 0)),
            pl.BlockSpec((VB, DIM), lambda i: (i, 0)),
        ],
        out_specs=pl.BlockSpec((BATCH, VB), lambda i: (0, i)),
        out_shape=jax.ShapeDtypeStruct((BATCH, N_NODE), jnp.float32),
    )(sel, g)


def kernel(item_embedding, pos_embedding, w_1, w_2, glu1_w, glu1_b, glu2_w,
           adj_values, items, inputs, alias_inputs, adj_indices):
    rows = adj_indices[0]
    cols = adj_indices[1]

    h = _sc_scatter(item_embedding, rows, cols, adj_values)

    gidx = jnp.take_along_axis(items, alias_inputs, axis=1).reshape(-1)
    seq = _sc_gather(h, gidx.astype(jnp.int32))

    pos_flip_pad = jnp.zeros((64, DIM), jnp.float32).at[0:SLEN].set(
        jnp.flip(pos_embedding, axis=0))
    sel = _attention(seq.reshape(BATCH, SLEN * DIM), inputs,
                     pos_flip_pad, w_1, glu1_w,
                     glu1_b.reshape(1, DIM), glu2_w, w_2.reshape(1, DIM))

    scores = _scores(sel, h)
    return scores, jnp.zeros((1,), jnp.float32)


# speedup vs baseline: 1.6585x; 1.0060x over previous
import functools

import jax
import jax.numpy as jnp
from jax import lax
from jax.experimental import pallas as pl
from jax.experimental.pallas import tpu as pltpu
from jax.experimental.pallas import tpu_sc as plsc

N_NODE = 50001
DIM = 128
E_EDGES = 800000
BATCH = 1024
SLEN = 50
W_K = 12.0

N_PAD = 50176
N_CHUNKS = 8
CHUNK = N_PAD // N_CHUNKS
TILE_ROWS = CHUNK // 16
EPT = E_EDGES // 16
EBLK = 2000
NBLK = EPT // EBLK
ZROWS = 56
CAP1D = 8320
GRP = 64


def _stage_start(rows, cols, vals, off, er, ec, ev, sem):
    pltpu.async_copy(rows.at[pl.ds(off, EBLK)], er, sem)
    pltpu.async_copy(cols.at[pl.ds(off, EBLK)], ec, sem)
    pltpu.async_copy(vals.at[pl.ds(off, EBLK)], ev, sem)


def _stage_wait(rows, cols, vals, off, er, ec, ev, sem):
    pltpu.make_async_copy(rows.at[pl.ds(off, EBLK)], er, sem).wait()
    pltpu.make_async_copy(cols.at[pl.ds(off, EBLK)], ec, sem).wait()
    pltpu.make_async_copy(vals.at[pl.ds(off, EBLK)], ev, sem).wait()


def _sc_scatter_body(emb, rows, cols, vals, h_out,
                     acc, er0, ec0, ev0, er1, ec1, ev1,
                     ccol, clrow, cval,
                     gbuf0, gbuf1, gbuf2, gbuf3, fbuf, idxb,
                     esem0, esem1, gsem0, gsem1, gsem2, gsem3, ssem0):
    c = lax.axis_index("c")
    s = lax.axis_index("s")
    tbase = s * EPT
    ebuf = ((er0, ec0, ev0, esem0), (er1, ec1, ev1, esem1))
    ztrue = jnp.ones((16,), jnp.bool_)
    zi = jnp.zeros((16,), jnp.int32)
    zf = jnp.zeros((16,), jnp.float32)

    def _scan_buf(er, ec, ev, cnt):
        def _grp(g, cnt):
            g16 = pl.multiple_of(g * 16, 16)
            r = er[pl.ds(g16, 16)]
            cix = ec[pl.ds(g16, 16)]
            v = ev[pl.ds(g16, 16)]
            m = jnp.logical_and(r >= lo, r < lo + CHUNK)
            plsc.store_compressed(cval.at[pl.ds(cnt, 16)], v, mask=m)
            plsc.store_compressed(ccol.at[pl.ds(cnt, 16)], cix, mask=m)
            plsc.store_compressed(clrow.at[pl.ds(cnt, 16)], r - lo, mask=m)
            return cnt + plsc.all_reduce_population_count(m)[0]
        return lax.fori_loop(0, EBLK // 16, _grp, cnt)

    def _gather_start(j, gb, gs):
        pltpu.async_copy(emb.at[ccol.at[pl.ds(j * GRP, GRP)]], gb, gs)

    def _gather_wait(j, gb, gs):
        pltpu.make_async_copy(emb.at[ccol.at[pl.ds(j * GRP, GRP)]], gb,
                              gs).wait()

    def _scatter_start(gb, ix, ss):
        pltpu.async_copy(gb, acc.at[ix.at[0]], ss, add=True)

    def _scatter_wait(gb, ix, ss):
        pltpu.make_async_copy(gb, acc.at[ix.at[0]], ss).wait()

    for k in range(N_CHUNKS // 2):
        chunk_id = c * (N_CHUNKS // 2) + k
        lo = chunk_id * CHUNK

        def _zb(r, _):
            for cc in range(8):
                fbuf[r, pl.ds(cc * 16, 16)] = zf
            return 0
        lax.fori_loop(0, ZROWS, _zb, 0)
        for i in range(TILE_ROWS // ZROWS):
            pltpu.sync_copy(fbuf.at[pl.ds(0, ZROWS)],
                            acc.at[pl.ds(s * TILE_ROWS + i * ZROWS, ZROWS)])
        plsc.subcore_barrier()

        _stage_start(rows, cols, vals, pl.multiple_of(tbase, 8), *ebuf[0])

        def _blkpair(bb, cnt):
            blk0 = bb * 2
            off0 = pl.multiple_of(tbase + blk0 * EBLK, 8)
            off1 = pl.multiple_of(tbase + (blk0 + 1) * EBLK, 8)
            off2 = pl.multiple_of(tbase + (blk0 + 2) * EBLK, 8)
            _stage_wait(rows, cols, vals, off0, *ebuf[0])
            _stage_start(rows, cols, vals, off1, *ebuf[1])
            cnt = _scan_buf(er0, ec0, ev0, cnt)
            _stage_wait(rows, cols, vals, off1, *ebuf[1])
            _stage_start(rows, cols, vals, off2, *ebuf[0])
            cnt = _scan_buf(er1, ec1, ev1, cnt)
            return cnt

        cnt = lax.fori_loop(0, (NBLK - 1) // 2, _blkpair, jnp.int32(0))
        lastoff = pl.multiple_of(tbase + (NBLK - 1) * EBLK, 8)
        _stage_wait(rows, cols, vals, lastoff, *ebuf[0])
        cnt = _scan_buf(er0, ec0, ev0, cnt)

        for t in range(GRP // 16):
            po = cnt + t * 16
            plsc.store_compressed(cval.at[pl.ds(po, 16)], zf, mask=ztrue)
            plsc.store_compressed(ccol.at[pl.ds(po, 16)], zi, mask=ztrue)
            plsc.store_compressed(clrow.at[pl.ds(po, 16)], zi, mask=ztrue)

        ng = (cnt + (GRP - 1)) // GRP

        gbs = ((gbuf0, gsem0), (gbuf1, gsem1), (gbuf2, gsem2),
               (gbuf3, gsem3))
        for pj in range(3):
            @pl.when(pj < ng)
            def _(pj=pj):
                _gather_start(pj, *gbs[pj])

        def _scale_fill(j, gb, ix):
            def _w(rq, _):
                voff = pl.multiple_of(j * GRP + rq * 16, 16)
                ix[0, pl.ds(rq * 16, 16)] = clrow[pl.ds(voff, 16)]
                vblk = cval[pl.ds(voff, 16)]
                for i in range(16):
                    vv = jnp.full((16,), vblk[i], jnp.float32)
                    rr = rq * 16 + i
                    for cc in range(8):
                        fbuf[rr, pl.ds(cc * 16, 16)] = (
                            gb[rr, pl.ds(cc * 16, 16)] * vv)
                return 0
            lax.fori_loop(0, GRP // 16, _w, 0)

        def _step(j, b):
            gb, gs = gbs[b]
            pb, ps = gbs[(b + 3) % 4]

            @pl.when(j + 3 < ng)
            def _():
                _gather_start(j + 3, pb, ps)

            _gather_wait(j, gb, gs)

            @pl.when(j >= 1)
            def _():
                _scatter_wait(fbuf, idxb, ssem0)

            _scale_fill(j, gb, idxb)
            _scatter_start(fbuf, idxb, ssem0)

        def _quad(jj, _):
            j0 = jj * 4
            for b in range(4):
                @pl.when(j0 + b < ng)
                def _(b=b):
                    _step(j0 + b, b)
            return 0

        lax.fori_loop(0, (ng + 3) // 4, _quad, 0)

        @pl.when(ng >= 1)
        def _():
            _scatter_wait(fbuf, idxb, ssem0)

        plsc.subcore_barrier()

        src = pl.multiple_of(s * TILE_ROWS, 8)
        dst = pl.multiple_of(lo + s * TILE_ROWS, 8)
        pltpu.sync_copy(acc.at[pl.ds(src, TILE_ROWS)],
                        h_out.at[pl.ds(dst, TILE_ROWS)])
        plsc.subcore_barrier()


_sc_scatter = functools.partial(
    pl.kernel,
    out_type=jax.ShapeDtypeStruct((N_PAD, DIM), jnp.float32),
    mesh=plsc.VectorSubcoreMesh(core_axis_name="c", subcore_axis_name="s"),
    scratch_types=[
        pltpu.VMEM_SHARED((CHUNK, DIM), jnp.float32),
        pltpu.VMEM((EBLK,), jnp.int32),
        pltpu.VMEM((EBLK,), jnp.int32),
        pltpu.VMEM((EBLK,), jnp.float32),
        pltpu.VMEM((EBLK,), jnp.int32),
        pltpu.VMEM((EBLK,), jnp.int32),
        pltpu.VMEM((EBLK,), jnp.float32),
        pltpu.VMEM((CAP1D,), jnp.int32),
        pltpu.VMEM((CAP1D,), jnp.int32),
        pltpu.VMEM((CAP1D,), jnp.float32),
        pltpu.VMEM((GRP, DIM), jnp.float32),
        pltpu.VMEM((GRP, DIM), jnp.float32),
        pltpu.VMEM((GRP, DIM), jnp.float32),
        pltpu.VMEM((GRP, DIM), jnp.float32),
        pltpu.VMEM((GRP, DIM), jnp.float32),
        pltpu.VMEM((1, GRP), jnp.int32),
        pltpu.SemaphoreType.DMA,
        pltpu.SemaphoreType.DMA,
        pltpu.SemaphoreType.DMA,
        pltpu.SemaphoreType.DMA,
        pltpu.SemaphoreType.DMA,
        pltpu.SemaphoreType.DMA,
        pltpu.SemaphoreType.DMA,
    ],
    compiler_params=pltpu.CompilerParams(needs_layout_passes=False),
)(_sc_scatter_body)


GPW = (BATCH * SLEN) // 32


def _sc_gather_body(g, gidx, out, gidxv, rbuf, sem):
    c = lax.axis_index("c")
    s = lax.axis_index("s")
    wid = s * 2 + c
    base = pl.multiple_of(wid * GPW, 8)
    pltpu.sync_copy(gidx.at[pl.ds(base, GPW)], gidxv)
    nfull = GPW // 128
    for k in range(nfull):
        pltpu.async_copy(g.at[gidxv.at[pl.ds(k * 128, 128)]], rbuf, sem).wait()
        pltpu.sync_copy(rbuf, out.at[pl.ds(base + k * 128, 128)])
    tail = GPW - nfull * 128
    if tail:
        pltpu.async_copy(g.at[gidxv.at[pl.ds(nfull * 128, tail)]],
                         rbuf.at[pl.ds(0, tail)], sem).wait()
        pltpu.sync_copy(rbuf.at[pl.ds(0, tail)],
                        out.at[pl.ds(base + nfull * 128, tail)])


_sc_gather = functools.partial(
    pl.kernel,
    out_type=jax.ShapeDtypeStruct((BATCH * SLEN, DIM), jnp.float32),
    mesh=plsc.VectorSubcoreMesh(core_axis_name="c", subcore_axis_name="s"),
    scratch_types=[
        pltpu.VMEM((GPW,), jnp.int32),
        pltpu.VMEM((128, DIM), jnp.float32),
        pltpu.SemaphoreType.DMA,
    ],
)(_sc_gather_body)


BB = 128


def _attn_body(seq_ref, inp_ref, pos_ref, w1_ref, g1w_ref, g1b_ref,
               g2w_ref, w2_ref, o_ref):
    seqh = seq_ref[...].reshape(BB, SLEN, DIM)
    snrm = jnp.sqrt(jnp.sum(seqh * seqh, axis=2, keepdims=True))
    seq = seqh / jnp.maximum(snrm, 1e-12)
    maskf = (inp_ref[...] != 0).astype(jnp.float32)
    msum = jnp.sum(maskf, axis=1, keepdims=True)
    hs = jnp.sum(seq * maskf[:, :, None], axis=1) / msum

    w1 = w1_ref[...]
    posw = jnp.dot(pos_ref[...], w1[0:DIM, :],
                   preferred_element_type=jnp.float32)[0:SLEN]
    sw = jnp.dot(seq.reshape(BB * SLEN, DIM), w1[DIM:2 * DIM, :],
                 preferred_element_type=jnp.float32).reshape(BB, SLEN, DIM)
    nh = jnp.tanh(posw[None, :, :] + sw)

    g2 = lax.dot_general(hs, g2w_ref[...], (((1,), (1,)), ((), ())),
                         preferred_element_type=jnp.float32)
    g1 = lax.dot_general(nh.reshape(BB * SLEN, DIM), g1w_ref[...],
                         (((1,), (1,)), ((), ())),
                         preferred_element_type=jnp.float32).reshape(
                             BB, SLEN, DIM)
    nh2 = jax.nn.sigmoid(g1 + g1b_ref[...][None, :, :] + g2[:, None, :])

    beta = jnp.sum(nh2 * w2_ref[...][None, :, :], axis=-1, keepdims=True)
    beta = beta * maskf[:, :, None]
    sess = jnp.sum(beta * seq, axis=1)
    nrm = jnp.sqrt(jnp.sum(sess * sess, axis=1, keepdims=True))
    o_ref[...] = W_K * sess / jnp.maximum(nrm, 1e-12)


def _attention(seq2d, inp, pos_flip_pad, w_1, glu1_w, glu1_b2, glu2_w, w_2r):
    zero2 = lambda i: (0, 0)
    return pl.pallas_call(
        _attn_body,
        grid=(BATCH // BB,),
        in_specs=[
            pl.BlockSpec((BB, SLEN * DIM), lambda i: (i, 0)),
            pl.BlockSpec((BB, SLEN), lambda i: (i, 0)),
            pl.BlockSpec((64, DIM), zero2),
            pl.BlockSpec((2 * DIM, DIM), zero2),
            pl.BlockSpec((DIM, DIM), zero2),
            pl.BlockSpec((1, DIM), zero2),
            pl.BlockSpec((DIM, DIM), zero2),
            pl.BlockSpec((1, DIM), zero2),
        ],
        out_specs=pl.BlockSpec((BB, DIM), lambda i: (i, 0)),
        out_shape=jax.ShapeDtypeStruct((BATCH, DIM), jnp.float32),
    )(seq2d, inp, pos_flip_pad, w_1, glu1_w, glu1_b2, glu2_w, w_2r)


VB = 2048


def _scores_body(sel_ref, g_ref, o_ref):
    hb = g_ref[...]
    n = jnp.sqrt(jnp.sum(hb * hb, axis=1, keepdims=True))
    hn = (hb / jnp.maximum(n, 1e-12)).astype(jnp.bfloat16)
    o_ref[...] = lax.dot_general(
        sel_ref[...].astype(jnp.bfloat16), hn, (((1,), (1,)), ((), ())),
        preferred_element_type=jnp.float32)


def _scores(sel, g):
    nblk = (N_NODE + VB - 1) // VB
    return pl.pallas_call(
        _scores_body,
        grid=(nblk,),
        in_specs=[
            pl.BlockSpec((BATCH, DIM), lambda i: (0, 0)),
            pl.BlockSpec((VB, DIM), lambda i: (i, 0)),
        ],
        out_specs=pl.BlockSpec((BATCH, VB), lambda i: (0, i)),
        out_shape=jax.ShapeDtypeStruct((BATCH, N_NODE), jnp.float32),
    )(sel, g)


def kernel(item_embedding, pos_embedding, w_1, w_2, glu1_w, glu1_b, glu2_w,
           adj_values, items, inputs, alias_inputs, adj_indices):
    rows = adj_indices[0]
    cols = adj_indices[1]

    h = _sc_scatter(item_embedding, rows, cols, adj_values)

    gidx = jnp.take_along_axis(items, alias_inputs, axis=1).reshape(-1)
    seq = _sc_gather(h, gidx.astype(jnp.int32))

    pos_flip_pad = jnp.zeros((64, DIM), jnp.float32).at[0:SLEN].set(
        jnp.flip(pos_embedding, axis=0))
    sel = _attention(seq.reshape(BATCH, SLEN * DIM), inputs,
                     pos_flip_pad, w_1, glu1_w,
                     glu1_b.reshape(1, DIM), glu2_w, w_2.reshape(1, DIM))

    scores = _scores(sel, h)
    return scores, jnp.zeros((1,), jnp.float32)


# speedup vs baseline: 1.6739x; 1.0093x over previous
import functools

import jax
import jax.numpy as jnp
from jax import lax
from jax.experimental import pallas as pl
from jax.experimental.pallas import tpu as pltpu
from jax.experimental.pallas import tpu_sc as plsc

N_NODE = 50001
DIM = 128
E_EDGES = 800000
BATCH = 1024
SLEN = 50
W_K = 12.0

N_PAD = 50176
N_CHUNKS = 8
CHUNK = N_PAD // N_CHUNKS
TILE_ROWS = CHUNK // 16
EPT = E_EDGES // 16
EBLK = 2000
NBLK = EPT // EBLK
ZROWS = 56
CAP1D = 8320
GRP = 64


def _stage_start(rows, cols, vals, off, er, ec, ev, sem):
    pltpu.async_copy(rows.at[pl.ds(off, EBLK)], er, sem)
    pltpu.async_copy(cols.at[pl.ds(off, EBLK)], ec, sem)
    pltpu.async_copy(vals.at[pl.ds(off, EBLK)], ev, sem)


def _stage_wait(rows, cols, vals, off, er, ec, ev, sem):
    pltpu.make_async_copy(rows.at[pl.ds(off, EBLK)], er, sem).wait()
    pltpu.make_async_copy(cols.at[pl.ds(off, EBLK)], ec, sem).wait()
    pltpu.make_async_copy(vals.at[pl.ds(off, EBLK)], ev, sem).wait()


def _sc_scatter_body(emb, rows, cols, vals, h_out,
                     acc, er0, ec0, ev0, er1, ec1, ev1,
                     ccol, clrow, cval,
                     gbuf0, gbuf1, gbuf2, gbuf3, fbuf, idxb,
                     esem0, esem1, gsem0, gsem1, gsem2, gsem3, ssem0):
    c = lax.axis_index("c")
    s = lax.axis_index("s")
    tbase = s * EPT
    ebuf = ((er0, ec0, ev0, esem0), (er1, ec1, ev1, esem1))
    ztrue = jnp.ones((16,), jnp.bool_)
    zi = jnp.zeros((16,), jnp.int32)
    zf = jnp.zeros((16,), jnp.float32)

    def _scan_buf(er, ec, ev, cnt):
        def _grp(g, cnt):
            g16 = pl.multiple_of(g * 16, 16)
            r = er[pl.ds(g16, 16)]
            cix = ec[pl.ds(g16, 16)]
            v = ev[pl.ds(g16, 16)]
            m = jnp.logical_and(r >= lo, r < lo + CHUNK)
            plsc.store_compressed(cval.at[pl.ds(cnt, 16)], v, mask=m)
            plsc.store_compressed(ccol.at[pl.ds(cnt, 16)], cix, mask=m)
            plsc.store_compressed(clrow.at[pl.ds(cnt, 16)], r - lo, mask=m)
            return cnt + plsc.all_reduce_population_count(m)[0]
        return lax.fori_loop(0, EBLK // 16, _grp, cnt)

    def _gather_start(j, gb, gs):
        pltpu.async_copy(emb.at[ccol.at[pl.ds(j * GRP, GRP)]], gb, gs)

    def _gather_wait(j, gb, gs):
        pltpu.make_async_copy(emb.at[ccol.at[pl.ds(j * GRP, GRP)]], gb,
                              gs).wait()

    def _scatter_start(gb, ix, ss):
        pltpu.async_copy(gb, acc.at[ix.at[0]], ss, add=True)

    def _scatter_wait(gb, ix, ss):
        pltpu.make_async_copy(gb, acc.at[ix.at[0]], ss).wait()

    for k in range(N_CHUNKS // 2):
        chunk_id = c * (N_CHUNKS // 2) + k
        lo = chunk_id * CHUNK

        def _zb(r, _):
            for cc in range(8):
                fbuf[r, pl.ds(cc * 16, 16)] = zf
            return 0
        lax.fori_loop(0, ZROWS, _zb, 0)
        for i in range(TILE_ROWS // ZROWS):
            pltpu.sync_copy(fbuf.at[pl.ds(0, ZROWS)],
                            acc.at[pl.ds(s * TILE_ROWS + i * ZROWS, ZROWS)])
        plsc.subcore_barrier()

        _stage_start(rows, cols, vals, pl.multiple_of(tbase, 8), *ebuf[0])

        def _blkpair(bb, cnt):
            blk0 = bb * 2
            off0 = pl.multiple_of(tbase + blk0 * EBLK, 8)
            off1 = pl.multiple_of(tbase + (blk0 + 1) * EBLK, 8)
            off2 = pl.multiple_of(tbase + (blk0 + 2) * EBLK, 8)
            _stage_wait(rows, cols, vals, off0, *ebuf[0])
            _stage_start(rows, cols, vals, off1, *ebuf[1])
            cnt = _scan_buf(er0, ec0, ev0, cnt)
            _stage_wait(rows, cols, vals, off1, *ebuf[1])
            _stage_start(rows, cols, vals, off2, *ebuf[0])
            cnt = _scan_buf(er1, ec1, ev1, cnt)
            return cnt

        cnt = lax.fori_loop(0, (NBLK - 1) // 2, _blkpair, jnp.int32(0))
        lastoff = pl.multiple_of(tbase + (NBLK - 1) * EBLK, 8)
        _stage_wait(rows, cols, vals, lastoff, *ebuf[0])
        cnt = _scan_buf(er0, ec0, ev0, cnt)

        for t in range(GRP // 16):
            po = cnt + t * 16
            plsc.store_compressed(cval.at[pl.ds(po, 16)], zf, mask=ztrue)
            plsc.store_compressed(ccol.at[pl.ds(po, 16)], zi, mask=ztrue)
            plsc.store_compressed(clrow.at[pl.ds(po, 16)], zi, mask=ztrue)

        ng = (cnt + (GRP - 1)) // GRP

        gbs = ((gbuf0, gsem0), (gbuf1, gsem1), (gbuf2, gsem2),
               (gbuf3, gsem3))
        for pj in range(3):
            @pl.when(pj < ng)
            def _(pj=pj):
                _gather_start(pj, *gbs[pj])

        def _scale_fill(j, gb, ix):
            def _w(rq, _):
                voff = pl.multiple_of(j * GRP + rq * 16, 16)
                ix[0, pl.ds(rq * 16, 16)] = clrow[pl.ds(voff, 16)]
                vblk = cval[pl.ds(voff, 16)]
                for i in range(16):
                    vv = jnp.full((16,), vblk[i], jnp.float32)
                    rr = rq * 16 + i
                    for cc in range(8):
                        fbuf[rr, pl.ds(cc * 16, 16)] = (
                            gb[rr, pl.ds(cc * 16, 16)] * vv)
                return 0
            lax.fori_loop(0, GRP // 16, _w, 0)

        def _step(j, b):
            gb, gs = gbs[b]
            pb, ps = gbs[(b + 3) % 4]

            @pl.when(j + 3 < ng)
            def _():
                _gather_start(j + 3, pb, ps)

            _gather_wait(j, gb, gs)

            @pl.when(j >= 1)
            def _():
                _scatter_wait(fbuf, idxb, ssem0)

            _scale_fill(j, gb, idxb)
            _scatter_start(fbuf, idxb, ssem0)

        def _quad(jj, _):
            j0 = jj * 4
            for b in range(4):
                @pl.when(j0 + b < ng)
                def _(b=b):
                    _step(j0 + b, b)
            return 0

        lax.fori_loop(0, (ng + 3) // 4, _quad, 0)

        @pl.when(ng >= 1)
        def _():
            _scatter_wait(fbuf, idxb, ssem0)

        plsc.subcore_barrier()

        src = pl.multiple_of(s * TILE_ROWS, 8)
        dst = pl.multiple_of(lo + s * TILE_ROWS, 8)
        pltpu.sync_copy(acc.at[pl.ds(src, TILE_ROWS)],
                        h_out.at[pl.ds(dst, TILE_ROWS)])
        plsc.subcore_barrier()


_sc_scatter = functools.partial(
    pl.kernel,
    out_type=jax.ShapeDtypeStruct((N_PAD, DIM), jnp.float32),
    mesh=plsc.VectorSubcoreMesh(core_axis_name="c", subcore_axis_name="s"),
    scratch_types=[
        pltpu.VMEM_SHARED((CHUNK, DIM), jnp.float32),
        pltpu.VMEM((EBLK,), jnp.int32),
        pltpu.VMEM((EBLK,), jnp.int32),
        pltpu.VMEM((EBLK,), jnp.float32),
        pltpu.VMEM((EBLK,), jnp.int32),
        pltpu.VMEM((EBLK,), jnp.int32),
        pltpu.VMEM((EBLK,), jnp.float32),
        pltpu.VMEM((CAP1D,), jnp.int32),
        pltpu.VMEM((CAP1D,), jnp.int32),
        pltpu.VMEM((CAP1D,), jnp.float32),
        pltpu.VMEM((GRP, DIM), jnp.float32),
        pltpu.VMEM((GRP, DIM), jnp.float32),
        pltpu.VMEM((GRP, DIM), jnp.float32),
        pltpu.VMEM((GRP, DIM), jnp.float32),
        pltpu.VMEM((GRP, DIM), jnp.float32),
        pltpu.VMEM((1, GRP), jnp.int32),
        pltpu.SemaphoreType.DMA,
        pltpu.SemaphoreType.DMA,
        pltpu.SemaphoreType.DMA,
        pltpu.SemaphoreType.DMA,
        pltpu.SemaphoreType.DMA,
        pltpu.SemaphoreType.DMA,
        pltpu.SemaphoreType.DMA,
    ],
    compiler_params=pltpu.CompilerParams(needs_layout_passes=False),
)(_sc_scatter_body)


GPW = (BATCH * SLEN) // 32


def _sc_gather_body(g, gidx, out, gidxv, rbuf0, rbuf1, sem0, sem1):
    c = lax.axis_index("c")
    s = lax.axis_index("s")
    wid = s * 2 + c
    base = pl.multiple_of(wid * GPW, 8)
    pltpu.sync_copy(gidx.at[pl.ds(base, GPW)], gidxv)
    nfull = GPW // 128
    tail = GPW - nfull * 128
    bufs = ((rbuf0, sem0), (rbuf1, sem1))
    nrounds = nfull + (1 if tail else 0)

    def _start(k):
        rb, sm = bufs[k % 2]
        if k < nfull:
            pltpu.async_copy(g.at[gidxv.at[pl.ds(k * 128, 128)]], rb, sm)
        else:
            pltpu.async_copy(g.at[gidxv.at[pl.ds(k * 128, tail)]],
                             rb.at[pl.ds(0, tail)], sm)

    def _finish(k):
        rb, sm = bufs[k % 2]
        if k < nfull:
            pltpu.make_async_copy(g.at[gidxv.at[pl.ds(k * 128, 128)]], rb,
                                  sm).wait()
            pltpu.sync_copy(rb, out.at[pl.ds(base + k * 128, 128)])
        else:
            pltpu.make_async_copy(g.at[gidxv.at[pl.ds(k * 128, tail)]],
                                  rb.at[pl.ds(0, tail)], sm).wait()
            pltpu.sync_copy(rb.at[pl.ds(0, tail)],
                            out.at[pl.ds(base + k * 128, tail)])

    _start(0)
    for k in range(nrounds):
        if k + 1 < nrounds:
            _start(k + 1)
        _finish(k)


_sc_gather = functools.partial(
    pl.kernel,
    out_type=jax.ShapeDtypeStruct((BATCH * SLEN, DIM), jnp.float32),
    mesh=plsc.VectorSubcoreMesh(core_axis_name="c", subcore_axis_name="s"),
    scratch_types=[
        pltpu.VMEM((GPW,), jnp.int32),
        pltpu.VMEM((128, DIM), jnp.float32),
        pltpu.VMEM((128, DIM), jnp.float32),
        pltpu.SemaphoreType.DMA,
        pltpu.SemaphoreType.DMA,
    ],
)(_sc_gather_body)


BB = 128


def _attn_body(seq_ref, inp_ref, pos_ref, w1_ref, g1w_ref, g1b_ref,
               g2w_ref, w2_ref, o_ref):
    seqh = seq_ref[...].reshape(BB, SLEN, DIM)
    snrm = jnp.sqrt(jnp.sum(seqh * seqh, axis=2, keepdims=True))
    seq = seqh / jnp.maximum(snrm, 1e-12)
    maskf = (inp_ref[...] != 0).astype(jnp.float32)
    msum = jnp.sum(maskf, axis=1, keepdims=True)
    hs = jnp.sum(seq * maskf[:, :, None], axis=1) / msum

    w1 = w1_ref[...]
    posw = jnp.dot(pos_ref[...], w1[0:DIM, :],
                   preferred_element_type=jnp.float32)[0:SLEN]
    sw = jnp.dot(seq.reshape(BB * SLEN, DIM), w1[DIM:2 * DIM, :],
                 preferred_element_type=jnp.float32).reshape(BB, SLEN, DIM)
    nh = jnp.tanh(posw[None, :, :] + sw)

    g2 = lax.dot_general(hs, g2w_ref[...], (((1,), (1,)), ((), ())),
                         preferred_element_type=jnp.float32)
    g1 = lax.dot_general(nh.reshape(BB * SLEN, DIM), g1w_ref[...],
                         (((1,), (1,)), ((), ())),
                         preferred_element_type=jnp.float32).reshape(
                             BB, SLEN, DIM)
    nh2 = jax.nn.sigmoid(g1 + g1b_ref[...][None, :, :] + g2[:, None, :])

    beta = jnp.sum(nh2 * w2_ref[...][None, :, :], axis=-1, keepdims=True)
    beta = beta * maskf[:, :, None]
    sess = jnp.sum(beta * seq, axis=1)
    nrm = jnp.sqrt(jnp.sum(sess * sess, axis=1, keepdims=True))
    o_ref[...] = W_K * sess / jnp.maximum(nrm, 1e-12)


def _attention(seq2d, inp, pos_flip_pad, w_1, glu1_w, glu1_b2, glu2_w, w_2r):
    zero2 = lambda i: (0, 0)
    return pl.pallas_call(
        _attn_body,
        grid=(BATCH // BB,),
        in_specs=[
            pl.BlockSpec((BB, SLEN * DIM), lambda i: (i, 0)),
            pl.BlockSpec((BB, SLEN), lambda i: (i, 0)),
            pl.BlockSpec((64, DIM), zero2),
            pl.BlockSpec((2 * DIM, DIM), zero2),
            pl.BlockSpec((DIM, DIM), zero2),
            pl.BlockSpec((1, DIM), zero2),
            pl.BlockSpec((DIM, DIM), zero2),
            pl.BlockSpec((1, DIM), zero2),
        ],
        out_specs=pl.BlockSpec((BB, DIM), lambda i: (i, 0)),
        out_shape=jax.ShapeDtypeStruct((BATCH, DIM), jnp.float32),
    )(seq2d, inp, pos_flip_pad, w_1, glu1_w, glu1_b2, glu2_w, w_2r)


VB = 2048


def _scores_body(sel_ref, g_ref, o_ref):
    hb = g_ref[...]
    n = jnp.sqrt(jnp.sum(hb * hb, axis=1, keepdims=True))
    hn = (hb / jnp.maximum(n, 1e-12)).astype(jnp.bfloat16)
    o_ref[...] = lax.dot_general(
        sel_ref[...].astype(jnp.bfloat16), hn, (((1,), (1,)), ((), ())),
        preferred_element_type=jnp.float32)


def _scores(sel, g):
    nblk = (N_NODE + VB - 1) // VB
    return pl.pallas_call(
        _scores_body,
        grid=(nblk,),
        in_specs=[
            pl.BlockSpec((BATCH, DIM), lambda i: (0, 0)),
            pl.BlockSpec((VB, DIM), lambda i: (i, 0)),
        ],
        out_specs=pl.BlockSpec((BATCH, VB), lambda i: (0, i)),
        out_shape=jax.ShapeDtypeStruct((BATCH, N_NODE), jnp.float32),
    )(sel, g)


def kernel(item_embedding, pos_embedding, w_1, w_2, glu1_w, glu1_b, glu2_w,
           adj_values, items, inputs, alias_inputs, adj_indices):
    rows = adj_indices[0]
    cols = adj_indices[1]

    h = _sc_scatter(item_embedding, rows, cols, adj_values)

    gidx = jnp.take_along_axis(items, alias_inputs, axis=1).reshape(-1)
    seq = _sc_gather(h, gidx.astype(jnp.int32))

    pos_flip_pad = jnp.zeros((64, DIM), jnp.float32).at[0:SLEN].set(
        jnp.flip(pos_embedding, axis=0))
    sel = _attention(seq.reshape(BATCH, SLEN * DIM), inputs,
                     pos_flip_pad, w_1, glu1_w,
                     glu1_b.reshape(1, DIM), glu2_w, w_2.reshape(1, DIM))

    scores = _scores(sel, h)
    return scores, jnp.zeros((1,), jnp.float32)
